# Initial kernel scaffold; baseline (speedup 1.0000x reference)
import jax
import jax.numpy as jnp
from jax.experimental import pallas as pl


def kernel(x_fix, x_train, word_mat, trained_table):
    raise NotImplementedError("write your pallas kernel here")


# speedup vs baseline: 5.9715x; 5.9715x over previous
import functools

import jax
import jax.numpy as jnp
from jax import lax
from jax.experimental import pallas as pl
from jax.experimental.pallas import tpu as pltpu
from jax.experimental.pallas import tpu_sc as plsc

VOCAB = 100000
D = 64
N = 4096 * 200

_INFO = plsc.get_sparse_core_info()
NC = _INFO.num_cores
NS = _INFO.num_subcores
LANES = _INFO.num_lanes
NW = NC * NS

PER_W = N // NW
ROWS = 512
CHUNKS = PER_W // ROWS


def _body(xf_hbm, xt_hbm, wm_hbm, tt_hbm, out_hbm,
          idxf_v, idxt_v, rows_a, rows_b, sem_a, sem_b):
    wid = lax.axis_index("s") * NC + lax.axis_index("c")
    base = pl.multiple_of(wid * PER_W, PER_W)

    pltpu.sync_copy(xf_hbm.at[pl.ds(base, PER_W)], idxf_v)
    pltpu.sync_copy(xt_hbm.at[pl.ds(base, PER_W)], idxt_v)

    def chunk_body(c, carry):
        off = pl.multiple_of(c * ROWS, ROWS)
        cp_a = pltpu.async_copy(wm_hbm.at[idxf_v.at[pl.ds(off, ROWS)]],
                                rows_a, sem_a)
        cp_b = pltpu.async_copy(tt_hbm.at[idxt_v.at[pl.ds(off, ROWS)]],
                                rows_b, sem_b)
        cp_a.wait()
        cp_b.wait()

        def add_row(r, carry2):
            for j in range(D // LANES):
                a = rows_a[r, pl.ds(j * LANES, LANES)]
                b = rows_b[r, pl.ds(j * LANES, LANES)]
                rows_a[r, pl.ds(j * LANES, LANES)] = a + b
            return carry2

        lax.fori_loop(0, ROWS, add_row, 0)
        pltpu.sync_copy(rows_a, out_hbm.at[pl.ds(base + off, ROWS)])
        return carry

    lax.fori_loop(0, CHUNKS, chunk_body, 0)


@jax.jit
def _dual_embed(xf, xt, wm, tt):
    mesh = plsc.VectorSubcoreMesh(core_axis_name="c", subcore_axis_name="s")
    f = functools.partial(
        pl.kernel,
        out_type=jax.ShapeDtypeStruct((N, D), jnp.float32),
        mesh=mesh,
        scratch_types=[
            pltpu.VMEM((PER_W,), jnp.int32),
            pltpu.VMEM((PER_W,), jnp.int32),
            pltpu.VMEM((ROWS, D), jnp.float32),
            pltpu.VMEM((ROWS, D), jnp.float32),
            pltpu.SemaphoreType.DMA,
            pltpu.SemaphoreType.DMA,
        ],
        compiler_params=pltpu.CompilerParams(use_tc_tiling_on_sc=False),
    )(_body)
    return f(xf, xt, wm, tt)


def kernel(x_fix, x_train, word_mat, trained_table):
    b, s = x_fix.shape
    xf = x_fix.reshape(-1).astype(jnp.int32)
    xt = x_train.reshape(-1).astype(jnp.int32)
    out = _dual_embed(xf, xt, word_mat, trained_table)
    return out.reshape(b, s, D)


# speedup vs baseline: 6.5135x; 1.0908x over previous
import functools

import jax
import jax.numpy as jnp
from jax import lax
from jax.experimental import pallas as pl
from jax.experimental.pallas import tpu as pltpu
from jax.experimental.pallas import tpu_sc as plsc

VOCAB = 100000
D = 64
N = 4096 * 200

_INFO = plsc.get_sparse_core_info()
NC = _INFO.num_cores
NS = _INFO.num_subcores
LANES = _INFO.num_lanes
NW = NC * NS

PER_W = N // NW
ROWS = 512
CHUNKS = PER_W // ROWS


def _body(xf_hbm, xt_hbm, wm_hbm, tt_hbm, out_hbm,
          idxf_v, idxt_v, rows_a, rows_b, sem_a, sem_b):
    wid = lax.axis_index("s") * NC + lax.axis_index("c")
    base = pl.multiple_of(wid * PER_W, PER_W)

    pltpu.sync_copy(xf_hbm.at[pl.ds(base, PER_W)], idxf_v)
    pltpu.sync_copy(xt_hbm.at[pl.ds(base, PER_W)], idxt_v)

    def chunk_body(c, carry):
        off = pl.multiple_of(c * ROWS, ROWS)
        cp_a = pltpu.async_copy(wm_hbm.at[idxf_v.at[pl.ds(off, ROWS)]],
                                rows_a, sem_a)
        cp_a.wait()
        cp_b = pltpu.async_copy(tt_hbm.at[idxt_v.at[pl.ds(off, ROWS)]],
                                rows_a, sem_b, add=True)
        cp_b.wait()
        pltpu.sync_copy(rows_a, out_hbm.at[pl.ds(base + off, ROWS)])
        return carry

    lax.fori_loop(0, CHUNKS, chunk_body, 0)


@jax.jit
def _dual_embed(xf, xt, wm, tt):
    mesh = plsc.VectorSubcoreMesh(core_axis_name="c", subcore_axis_name="s")
    f = functools.partial(
        pl.kernel,
        out_type=jax.ShapeDtypeStruct((N, D), jnp.float32),
        mesh=mesh,
        scratch_types=[
            pltpu.VMEM((PER_W,), jnp.int32),
            pltpu.VMEM((PER_W,), jnp.int32),
            pltpu.VMEM((ROWS, D), jnp.float32),
            pltpu.VMEM((ROWS, D), jnp.float32),
            pltpu.SemaphoreType.DMA,
            pltpu.SemaphoreType.DMA,
        ],
        compiler_params=pltpu.CompilerParams(use_tc_tiling_on_sc=False),
    )(_body)
    return f(xf, xt, wm, tt)


def kernel(x_fix, x_train, word_mat, trained_table):
    b, s = x_fix.shape
    xf = x_fix.reshape(-1).astype(jnp.int32)
    xt = x_train.reshape(-1).astype(jnp.int32)
    out = _dual_embed(xf, xt, word_mat, trained_table)
    return out.reshape(b, s, D)


# speedup vs baseline: 6.9500x; 1.0670x over previous
import functools

import jax
import jax.numpy as jnp
from jax import lax
from jax.experimental import pallas as pl
from jax.experimental.pallas import tpu as pltpu
from jax.experimental.pallas import tpu_sc as plsc

VOCAB = 100000
D = 64
N = 4096 * 200

_INFO = plsc.get_sparse_core_info()
NC = _INFO.num_cores
NS = _INFO.num_subcores
LANES = _INFO.num_lanes
NW = NC * NS

PER_W = N // NW
ROWS = 256
NBUF = 4
CHUNKS = PER_W // ROWS
GROUPS = CHUNKS // NBUF


def _body(xf_hbm, xt_hbm, wm_hbm, tt_hbm, out_hbm,
          idxf_v, idxt_v, rows_v, sems):
    wid = lax.axis_index("s") * NC + lax.axis_index("c")
    base = pl.multiple_of(wid * PER_W, PER_W)

    pltpu.sync_copy(xf_hbm.at[pl.ds(base, PER_W)], idxf_v)
    pltpu.sync_copy(xt_hbm.at[pl.ds(base, PER_W)], idxt_v)

    def gather_a(g, b):
        off = pl.multiple_of((g * NBUF + b) * ROWS, ROWS)
        return pltpu.make_async_copy(wm_hbm.at[idxf_v.at[pl.ds(off, ROWS)]],
                                     rows_v.at[b], sems.at[b])

    def gather_b(g, b):
        off = pl.multiple_of((g * NBUF + b) * ROWS, ROWS)
        return pltpu.make_async_copy(tt_hbm.at[idxt_v.at[pl.ds(off, ROWS)]],
                                     rows_v.at[b], sems.at[b])

    def scatter_out(g, b):
        off = pl.multiple_of((g * NBUF + b) * ROWS, ROWS)
        return pltpu.make_async_copy(rows_v.at[b],
                                     out_hbm.at[pl.ds(base + off, ROWS)],
                                     sems.at[b])

    def group_body(g, carry):
        for b in range(NBUF):
            @pl.when(g > 0)
            def _wait_prev():
                scatter_out(g - 1, b).wait()
            gather_a(g, b).start()
        for b in range(NBUF):
            gather_a(g, b).wait()
            gather_b(g, b).start(add=True)
        for b in range(NBUF):
            gather_b(g, b).wait()
            scatter_out(g, b).start()
        return carry

    lax.fori_loop(0, GROUPS, group_body, 0)
    for b in range(NBUF):
        scatter_out(GROUPS - 1, b).wait()


@jax.jit
def _dual_embed(xf, xt, wm, tt):
    mesh = plsc.VectorSubcoreMesh(core_axis_name="c", subcore_axis_name="s")
    f = functools.partial(
        pl.kernel,
        out_type=jax.ShapeDtypeStruct((N, D), jnp.float32),
        mesh=mesh,
        scratch_types=[
            pltpu.VMEM((PER_W,), jnp.int32),
            pltpu.VMEM((PER_W,), jnp.int32),
            pltpu.VMEM((NBUF, ROWS, D), jnp.float32),
            pltpu.SemaphoreType.DMA((NBUF,)),
        ],
        compiler_params=pltpu.CompilerParams(use_tc_tiling_on_sc=False),
    )(_body)
    return f(xf, xt, wm, tt)


def kernel(x_fix, x_train, word_mat, trained_table):
    b, s = x_fix.shape
    xf = x_fix.reshape(-1).astype(jnp.int32)
    xt = x_train.reshape(-1).astype(jnp.int32)
    out = _dual_embed(xf, xt, word_mat, trained_table)
    return out.reshape(b, s, D)


# speedup vs baseline: 6.9536x; 1.0005x over previous
import functools

import jax
import jax.numpy as jnp
from jax import lax
from jax.experimental import pallas as pl
from jax.experimental.pallas import tpu as pltpu
from jax.experimental.pallas import tpu_sc as plsc

VOCAB = 100000
D = 64
B = 4096
S = 200

_INFO = plsc.get_sparse_core_info()
NC = _INFO.num_cores
NS = _INFO.num_subcores
NW = NC * NS

B_W = B // NW
NBUF = 4
GROUPS = B_W // NBUF


def _body(xf_hbm, xt_hbm, wm_hbm, tt_hbm, out_hbm, idxf_v, idxt_v, rows_v, sems):
    wid = lax.axis_index("s") * NC + lax.axis_index("c")
    base = pl.multiple_of(wid * B_W, B_W)

    pltpu.sync_copy(xf_hbm.at[pl.ds(base, B_W)], idxf_v)
    pltpu.sync_copy(xt_hbm.at[pl.ds(base, B_W)], idxt_v)

    def gather_a(g, b):
        return pltpu.make_async_copy(wm_hbm.at[idxf_v.at[g * NBUF + b]],
                                     rows_v.at[b], sems.at[b])

    def gather_b(g, b):
        return pltpu.make_async_copy(tt_hbm.at[idxt_v.at[g * NBUF + b]],
                                     rows_v.at[b], sems.at[b])

    def scatter_out(g, b):
        return pltpu.make_async_copy(rows_v.at[b],
                                     out_hbm.at[base + g * NBUF + b],
                                     sems.at[b])

    def group_body(g, carry):
        for b in range(NBUF):
            @pl.when(g > 0)
            def _wait_prev():
                scatter_out(g - 1, b).wait()
            gather_a(g, b).start()
        for b in range(NBUF):
            gather_a(g, b).wait()
            gather_b(g, b).start(add=True)
        for b in range(NBUF):
            gather_b(g, b).wait()
            scatter_out(g, b).start()
        return carry

    lax.fori_loop(0, GROUPS, group_body, 0)
    for b in range(NBUF):
        scatter_out(GROUPS - 1, b).wait()


@jax.jit
def _dual_embed(xf, xt, wm, tt):
    mesh = plsc.VectorSubcoreMesh(core_axis_name="c", subcore_axis_name="s")
    f = functools.partial(
        pl.kernel,
        out_type=jax.ShapeDtypeStruct((B, S, D), jnp.float32),
        mesh=mesh,
        scratch_types=[
            pltpu.VMEM((B_W, S), jnp.int32),
            pltpu.VMEM((B_W, S), jnp.int32),
            pltpu.VMEM((NBUF, S, D), jnp.float32),
            pltpu.SemaphoreType.DMA((NBUF,)),
        ],
        compiler_params=pltpu.CompilerParams(use_tc_tiling_on_sc=False),
    )(_body)
    return f(xf, xt, wm, tt)


def kernel(x_fix, x_train, word_mat, trained_table):
    return _dual_embed(x_fix.astype(jnp.int32), x_train.astype(jnp.int32),
                       word_mat, trained_table)
